# baseline (speedup 1.0000x reference)
import functools

import jax
import jax.numpy as jnp
from jax import lax
from jax.experimental import pallas as pl
from jax.experimental.pallas import tpu as pltpu
from jax.experimental.pallas import tpu_sc as plsc

BATCH = 16384
EMB_DIM = 4096
NUM_CLASSES = 1000
NUM_CORES = 2
NUM_SUBCORES = 16
NUM_TILES = NUM_CORES * NUM_SUBCORES
BPW = BATCH // NUM_TILES
NCACHE = 15
K = 8
CHW = K * EMB_DIM
UNROLL = 8
ADD_ITERS = EMB_DIM // 16 // UNROLL


def kernel(condition, image_emb, emb_table):
    cond = condition.astype(jnp.int32)
    tile_of = (cond * NUM_TILES) // NUM_CLASSES
    lo = (tile_of * NUM_CLASSES) // NUM_TILES
    hit = (cond - lo) < NCACHE
    key = tile_of * 2 + jnp.where(hit, 0, 1)
    pos = jnp.argsort(key, stable=True).astype(jnp.int32)
    cond_p = jnp.take(cond, pos, axis=0)

    mesh = plsc.VectorSubcoreMesh(core_axis_name="c", subcore_axis_name="s")

    @functools.partial(
        pl.kernel,
        mesh=mesh,
        out_type=jax.ShapeDtypeStruct((BATCH, EMB_DIM), jnp.float32),
        scratch_types=[
            pltpu.VMEM((NCACHE * EMB_DIM,), jnp.float32),
            pltpu.VMEM((BPW,), jnp.int32),
            pltpu.VMEM((BPW,), jnp.int32),
            pltpu.VMEM((BPW,), jnp.int32),
            pltpu.VMEM((K, EMB_DIM), jnp.float32),
            pltpu.VMEM((K, EMB_DIM), jnp.float32),
            pltpu.SemaphoreType.DMA,
            pltpu.SemaphoreType.DMA,
            pltpu.SemaphoreType.DMA,
        ],
    )
    def run(cond_hbm, pos_hbm, img_hbm, table_hbm, tflat_hbm, out_hbm,
            cache_v, idx_v, ivl_v, pos_v, rows_v, img_v,
            sem_g, sem_i, sem_o):
        cid = lax.axis_index("c")
        sid = lax.axis_index("s")
        wid = cid * NUM_SUBCORES + sid
        lo_t = (wid * NUM_CLASSES) // NUM_TILES

        pltpu.sync_copy(
            tflat_hbm.at[pl.ds(lo_t * EMB_DIM, NCACHE * EMB_DIM)], cache_v
        )
        base = wid * BPW
        pltpu.sync_copy(cond_hbm.at[pl.ds(base, BPW)], idx_v)
        pltpu.sync_copy(pos_hbm.at[pl.ds(base, BPW)], pos_v)

        lo_v = lax.broadcast_in_dim(lo_t, (16,), ())

        def loc_body(t, carry):
            sl = pl.ds(t * 16, 16)
            ivl_v[sl] = idx_v[sl] - lo_v
            return carry

        lax.fori_loop(0, BPW // 16, loc_body, 0)

        def drain(sem, buf):
            pltpu.make_async_copy(
                table_hbm.at[pl.ds(0, K)], buf, sem
            ).wait()

        def group_body(g, carry):
            iv = idx_v[pl.ds(g * 16, 16)]
            for half in range(2):
                j = g * 2 + half
                if half == 1:
                    drain(sem_o, img_v)
                else:
                    @pl.when(g > 0)
                    def _():
                        drain(sem_o, img_v)

                pltpu.async_copy(
                    img_hbm.at[pos_v.at[pl.ds(j * K, K)]], img_v, sem_i
                )

                l_first = iv[half * K] - lo_t
                l_last = iv[half * K + 7] - lo_t
                all_hit = jnp.logical_and(
                    jnp.logical_and(l_first >= 0, l_first < NCACHE),
                    jnp.logical_and(l_last >= 0, l_last < NCACHE),
                )

                locals_u = [iv[half * K + u] - lo_t for u in range(K)]

                def hit_mid(locals_u=locals_u):
                    pltpu.make_async_copy(
                        table_hbm.at[pl.ds(0, K)], img_v, sem_i
                    ).wait()
                    for u in range(K):
                        crow = cache_v.at[pl.ds(locals_u[u] * EMB_DIM, EMB_DIM)]

                        def ab(t, cc, u=u, crow=crow):
                            for uu in range(UNROLL):
                                off = (t * UNROLL + uu) * 16
                                img_v[u, pl.ds(off, 16)] = (
                                    img_v[u, pl.ds(off, 16)]
                                    + crow[pl.ds(off, 16)]
                                )
                            return cc

                        lax.fori_loop(0, ADD_ITERS, ab, 0)

                def miss_mid(j=j):
                    pltpu.async_copy(
                        table_hbm.at[idx_v.at[pl.ds(j * K, K)]], rows_v, sem_g
                    )
                    drain(sem_g, rows_v)
                    pltpu.make_async_copy(
                        table_hbm.at[pl.ds(0, K)], img_v, sem_i
                    ).wait()
                    for u in range(K):
                        def ab(t, cc, u=u):
                            for uu in range(UNROLL):
                                off = (t * UNROLL + uu) * 16
                                img_v[u, pl.ds(off, 16)] = (
                                    img_v[u, pl.ds(off, 16)]
                                    + rows_v[u, pl.ds(off, 16)]
                                )
                            return cc

                        lax.fori_loop(0, ADD_ITERS, ab, 0)

                lax.cond(all_hit, hit_mid, miss_mid)

                pltpu.async_copy(
                    img_v, out_hbm.at[pos_v.at[pl.ds(j * K, K)]], sem_o
                )
            return carry

        lax.fori_loop(0, BPW // 16, group_body, 0)
        drain(sem_o, img_v)

    return run(cond_p, pos, image_emb, emb_table, emb_table.reshape(-1))


# speedup vs baseline: 1.7681x; 1.7681x over previous
import functools

import jax
import jax.numpy as jnp
from jax import lax
from jax.experimental import pallas as pl
from jax.experimental.pallas import tpu as pltpu
from jax.experimental.pallas import tpu_sc as plsc

BATCH = 16384
EMB_DIM = 4096
NUM_CORES = 2
NUM_SUBCORES = 16
NUM_WORKERS = NUM_CORES * NUM_SUBCORES
BPW = BATCH // NUM_WORKERS
K = 8
CHW = K * EMB_DIM
UNROLL = 8
ADD_ITERS = EMB_DIM // 16 // UNROLL


def kernel(condition, image_emb, emb_table):
    mesh = plsc.VectorSubcoreMesh(core_axis_name="c", subcore_axis_name="s")

    @functools.partial(
        pl.kernel,
        mesh=mesh,
        out_type=jax.ShapeDtypeStruct((BATCH, EMB_DIM), jnp.float32),
        scratch_types=[
            pltpu.VMEM((BPW,), jnp.int32),
            pltpu.VMEM((K, EMB_DIM), jnp.float32),
            pltpu.VMEM((K, EMB_DIM), jnp.float32),
            pltpu.VMEM((K, EMB_DIM), jnp.float32),
            pltpu.SemaphoreType.DMA,
            pltpu.SemaphoreType.DMA,
            pltpu.SemaphoreType.DMA,
            pltpu.SemaphoreType.DMA,
        ],
    )
    def run(cond_hbm, img_hbm, table_hbm, out_hbm,
            idx_v, rows0, rows1, img_v, sem_g, sem_i, sem_o0, sem_o1):
        wid = lax.axis_index("s") * NUM_CORES + lax.axis_index("c")
        base = wid * BPW
        pltpu.sync_copy(cond_hbm.at[pl.ds(base, BPW)], idx_v)

        rows_bufs = (rows0, rows1)
        out_sems = (sem_o0, sem_o1)

        def drain(sem, buf):
            pltpu.make_async_copy(table_hbm.at[pl.ds(0, K)], buf, sem).wait()

        def group_body(g, carry):
            for half in range(2):
                j = g * 2 + half
                rows_b = rows_bufs[half]
                sem_o = out_sems[half]
                start = base + j * K

                @pl.when(g > 0)
                def _(rows_b=rows_b, sem_o=sem_o):
                    drain(sem_o, rows_b)

                gth = pltpu.async_copy(
                    table_hbm.at[idx_v.at[pl.ds(j * K, K)]], rows_b, sem_g
                )
                im = pltpu.async_copy(
                    img_hbm.at[pl.ds(start, K)], img_v, sem_i
                )
                gth.wait()
                im.wait()

                for r in range(K):
                    def add_body(t, c2, r=r, rows_b=rows_b):
                        for uu in range(UNROLL):
                            sl = pl.ds((t * UNROLL + uu) * 16, 16)
                            rows_b[r, sl] = rows_b[r, sl] + img_v[r, sl]
                        return c2

                    lax.fori_loop(0, ADD_ITERS, add_body, 0)

                pltpu.async_copy(rows_b, out_hbm.at[pl.ds(start, K)], sem_o)
            return carry

        lax.fori_loop(0, BPW // 16, group_body, 0)
        drain(sem_o0, rows0)
        drain(sem_o1, rows1)

    return run(condition, image_emb, emb_table)
